# baseline (speedup 1.0000x reference)
import jax
import jax.numpy as jnp
from jax import lax
from jax.experimental import pallas as pl
from jax.experimental.pallas import tpu as pltpu
from jax.experimental.pallas import tpu_sc as plsc

NC = 2
NS = 16
L = 16

B = 8
R = 2048
D = 512
N = R * D
NBINS = 4096
WPB = (NC * NS) // B
BPC = NS // WPB
ROWS_W = R // WPB
CR = 8
C = CR * D
NCHUNK = ROWS_W // CR
UNROLL = 8
GPR = D // L


def _sc_body(src_hbm, lab_hbm, out_hbm,
             sum_v, cnt_v, mean_v, red_s,
             lab0, lab1, dat0, dat1, ones_v,
             shared_s, cnt_sh0, cnt_sh1, cnt_sh2, cnt_sh3,
             lsem0, lsem1, ssem0, ssem1, csem0, csem1):
    c = lax.axis_index("c")
    s = lax.axis_index("s")
    bi = s // WPB
    batch = c * BPC + bi
    quarter = s % WPB
    row_base = quarter * ROWS_W

    bufs = ((lab0, dat0, lsem0, ssem0, csem0),
            (lab1, dat1, lsem1, ssem1, csem1))
    cnt_shs = (cnt_sh0, cnt_sh1, cnt_sh2, cnt_sh3)
    zeros = jnp.zeros((L,), jnp.float32)
    ones = jnp.ones((L,), jnp.float32)

    def chunk_rows(i):
        return pl.ds(pl.multiple_of(row_base + i * CR, CR), CR)

    def start_lab(i, labv, lsem):
        r0 = row_base + i * CR
        for r in range(CR):
            pltpu.async_copy(lab_hbm.at[batch, r0 + r],
                             labv.at[pl.ds(r * D, D)], lsem)

    def wait_lab(labv, lsem):
        for r in range(CR):
            pltpu.make_async_copy(lab_hbm.at[batch, row_base],
                                  labv.at[pl.ds(r * D, D)], lsem).wait()

    def zbody(i, _):
        for u in range(UNROLL):
            ds = pl.ds(i * (L * UNROLL) + u * L, L)
            sum_v[ds] = zeros
            cnt_v[ds] = zeros
        return _
    lax.fori_loop(0, NBINS // (L * UNROLL), zbody, None)

    def obody(j, _):
        for u in range(UNROLL):
            ones_v[pl.ds(j * (L * UNROLL) + u * L, L)] = ones
        return _
    lax.fori_loop(0, C // (L * UNROLL), obody, None)

    @pl.when(s == 0)
    def _zero_counts():
        for k in range(BPC):
            pltpu.sync_copy(cnt_v, cnt_shs[k])
    plsc.subcore_barrier()

    start_lab(0, lab0, lsem0)
    pltpu.async_copy(src_hbm.at[batch, chunk_rows(0)], dat0, lsem0)
    start_lab(1, lab1, lsem1)
    pltpu.async_copy(src_hbm.at[batch, chunk_rows(1)], dat1, lsem1)

    def chunk_pair1(g, _):
        for b, (labv, datv, lsem, _ssem, csem) in enumerate(bufs):
            cur = g * 2 + b
            wait_lab(labv, lsem)
            pltpu.make_async_copy(src_hbm.at[batch, chunk_rows(0)], datv, lsem).wait()

            for k in range(BPC):
                @pl.when(bi == k)
                def _fire_cnt(k=k):
                    pltpu.async_copy(ones_v, cnt_shs[k].at[labv], csem,
                                     add=True)

            def inner(j, _):
                r = j // (GPR // UNROLL)
                q = (j % (GPR // UNROLL)) * (L * UNROLL)
                labs = [labv[pl.ds(j * (L * UNROLL) + u * L, L)]
                        for u in range(UNROLL)]
                xs = [datv[r, pl.ds(q + u * L, L)] for u in range(UNROLL)]
                for u in range(UNROLL):
                    plsc.addupdate_scatter(sum_v, [labs[u]], xs[u])
                return _
            lax.fori_loop(0, C // (L * UNROLL), inner, None)

            for k in range(BPC):
                @pl.when(bi == k)
                def _drain_cnt(k=k):
                    pltpu.make_async_copy(ones_v, cnt_shs[k].at[labv],
                                          csem).wait()

            nxt = cur + 2

            @pl.when(nxt < NCHUNK)
            def _start_next():
                start_lab(nxt, labv, lsem)
                pltpu.async_copy(src_hbm.at[batch, chunk_rows(nxt)], datv, lsem)
        return _
    lax.fori_loop(0, NCHUNK // 2, chunk_pair1, None)

    pltpu.sync_copy(sum_v, shared_s.at[s])
    plsc.subcore_barrier()
    row0 = bi * WPB
    pltpu.sync_copy(shared_s.at[pl.ds(row0, WPB)], red_s)
    for k in range(BPC):
        @pl.when(bi == k)
        def _fetch_cnt(k=k):
            pltpu.sync_copy(cnt_shs[k], cnt_v)

    def mbody(i, _):
        for u in range(UNROLL):
            ds = pl.ds(i * (L * UNROLL) + u * L, L)
            ssum = red_s[0, ds] + red_s[1, ds] + red_s[2, ds] + red_s[3, ds]
            mean_v[ds] = ssum / cnt_v[ds]
        return _
    lax.fori_loop(0, NBINS // (L * UNROLL), mbody, None)

    start_lab(0, lab0, lsem0)
    start_lab(1, lab1, lsem1)

    def chunk_pair2(g, _):
        for b, (labv, datv, lsem, ssem, _csem) in enumerate(bufs):
            cur = g * 2 + b
            wait_lab(labv, lsem)

            @pl.when(cur >= 2)
            def _drain_store():
                pltpu.make_async_copy(datv, out_hbm.at[batch, chunk_rows(0)], ssem).wait()

            def inner(j, _):
                r = j // (GPR // UNROLL)
                q = (j % (GPR // UNROLL)) * (L * UNROLL)
                labs = [labv[pl.ds(j * (L * UNROLL) + u * L, L)]
                        for u in range(UNROLL)]
                gs = [plsc.load_gather(mean_v, [labs[u]])
                      for u in range(UNROLL)]
                for u in range(UNROLL):
                    datv[r, pl.ds(q + u * L, L)] = gs[u]
                return _
            lax.fori_loop(0, C // (L * UNROLL), inner, None)

            pltpu.async_copy(datv, out_hbm.at[batch, chunk_rows(cur)], ssem)

            nxt = cur + 2

            @pl.when(nxt < NCHUNK)
            def _start_next():
                start_lab(nxt, labv, lsem)
        return _
    lax.fori_loop(0, NCHUNK // 2, chunk_pair2, None)

    pltpu.make_async_copy(dat0, out_hbm.at[batch, chunk_rows(0)], ssem0).wait()
    pltpu.make_async_copy(dat1, out_hbm.at[batch, chunk_rows(0)], ssem1).wait()


@jax.jit
def kernel(src, labels):
    labels = labels.astype(jnp.int32)
    mesh = plsc.VectorSubcoreMesh(
        core_axis_name="c", subcore_axis_name="s",
        num_cores=NC, num_subcores=NS)
    out = pl.kernel(
        _sc_body,
        out_type=jax.ShapeDtypeStruct((B, R, D), jnp.float32),
        mesh=mesh,
        compiler_params=pltpu.CompilerParams(needs_layout_passes=False),
        scratch_types=[
            pltpu.VMEM((NBINS,), jnp.float32),
            pltpu.VMEM((NBINS,), jnp.float32),
            pltpu.VMEM((NBINS,), jnp.float32),
            pltpu.VMEM((WPB, NBINS), jnp.float32),
            pltpu.VMEM((C,), jnp.int32),
            pltpu.VMEM((C,), jnp.int32),
            pltpu.VMEM((CR, D), jnp.float32),
            pltpu.VMEM((CR, D), jnp.float32),
            pltpu.VMEM((C,), jnp.float32),
            pltpu.VMEM_SHARED((NS, NBINS), jnp.float32),
            pltpu.VMEM_SHARED((NBINS,), jnp.float32),
            pltpu.VMEM_SHARED((NBINS,), jnp.float32),
            pltpu.VMEM_SHARED((NBINS,), jnp.float32),
            pltpu.VMEM_SHARED((NBINS,), jnp.float32),
            pltpu.SemaphoreType.DMA,
            pltpu.SemaphoreType.DMA,
            pltpu.SemaphoreType.DMA,
            pltpu.SemaphoreType.DMA,
            pltpu.SemaphoreType.DMA,
            pltpu.SemaphoreType.DMA,
        ],
    )(src, labels)
    return out


# speedup vs baseline: 1.2491x; 1.2491x over previous
import jax
import jax.numpy as jnp
from jax import lax
from jax.experimental import pallas as pl
from jax.experimental.pallas import tpu as pltpu
from jax.experimental.pallas import tpu_sc as plsc

NC = 2
NS = 16
L = 16

B = 8
R = 2048
D = 512
N = R * D
NBINS = 4096
WPB = (NC * NS) // B
ROWS_W = R // WPB
CR = 16
C = CR * D
NCHUNK = ROWS_W // CR
UNROLL = 8
GPR = D // L


def _sc_body(src_hbm, lab_hbm, out_hbm,
             sum_v, cnt_v, mean_v, red_s, red_c,
             lab0, lab1, dat0, dat1, shared_s, shared_c,
             lsem0, lsem1, ssem0, ssem1):
    c = lax.axis_index("c")
    s = lax.axis_index("s")
    batch = c * (NS // WPB) + s // WPB
    quarter = s % WPB
    row_base = quarter * ROWS_W

    bufs = ((lab0, dat0, lsem0, ssem0), (lab1, dat1, lsem1, ssem1))
    zeros = jnp.zeros((L,), jnp.float32)
    ones = jnp.ones((L,), jnp.float32)

    def chunk_rows(i):
        return pl.ds(pl.multiple_of(row_base + i * CR, CR), CR)

    def zbody(i, _):
        for u in range(UNROLL):
            ds = pl.ds(i * (L * UNROLL) + u * L, L)
            sum_v[ds] = zeros
            cnt_v[ds] = zeros
        return _
    lax.fori_loop(0, NBINS // (L * UNROLL), zbody, None)

    pltpu.async_copy(lab_hbm.at[batch, chunk_rows(0)], lab0, lsem0)
    pltpu.async_copy(src_hbm.at[batch, chunk_rows(0)], dat0, lsem0)
    pltpu.async_copy(lab_hbm.at[batch, chunk_rows(1)], lab1, lsem1)
    pltpu.async_copy(src_hbm.at[batch, chunk_rows(1)], dat1, lsem1)

    def chunk_pair1(g, _):
        for b, (labv, datv, lsem, _ssem) in enumerate(bufs):
            cur = g * 2 + b
            pltpu.make_async_copy(lab_hbm.at[batch, chunk_rows(0)], labv, lsem).wait()
            pltpu.make_async_copy(src_hbm.at[batch, chunk_rows(0)], datv, lsem).wait()

            def inner(j, _):
                r = j // (GPR // UNROLL)
                q = (j % (GPR // UNROLL)) * (L * UNROLL)
                labs = [labv[r, pl.ds(q + u * L, L)] for u in range(UNROLL)]
                xs = [datv[r, pl.ds(q + u * L, L)] for u in range(UNROLL)]
                for u in range(UNROLL):
                    plsc.addupdate_scatter(sum_v, [labs[u]], xs[u])
                    plsc.addupdate_scatter(cnt_v, [labs[u]], ones)
                return _
            lax.fori_loop(0, C // (L * UNROLL), inner, None)

            nxt = cur + 2

            @pl.when(nxt < NCHUNK)
            def _start_next():
                pltpu.async_copy(lab_hbm.at[batch, chunk_rows(nxt)], labv, lsem)
                pltpu.async_copy(src_hbm.at[batch, chunk_rows(nxt)], datv, lsem)
        return _
    lax.fori_loop(0, NCHUNK // 2, chunk_pair1, None)

    pltpu.sync_copy(sum_v, shared_s.at[s])
    pltpu.sync_copy(cnt_v, shared_c.at[s])
    plsc.subcore_barrier()
    row0 = (s // WPB) * WPB
    pltpu.sync_copy(shared_s.at[pl.ds(row0, WPB)], red_s)
    pltpu.sync_copy(shared_c.at[pl.ds(row0, WPB)], red_c)

    def mbody(i, _):
        for u in range(UNROLL):
            ds = pl.ds(i * (L * UNROLL) + u * L, L)
            ssum = red_s[0, ds] + red_s[1, ds] + red_s[2, ds] + red_s[3, ds]
            scnt = red_c[0, ds] + red_c[1, ds] + red_c[2, ds] + red_c[3, ds]
            mean_v[ds] = ssum / scnt
        return _
    lax.fori_loop(0, NBINS // (L * UNROLL), mbody, None)

    pltpu.async_copy(lab_hbm.at[batch, chunk_rows(0)], lab0, lsem0)
    pltpu.async_copy(lab_hbm.at[batch, chunk_rows(1)], lab1, lsem1)

    def chunk_pair2(g, _):
        for b, (labv, datv, lsem, ssem) in enumerate(bufs):
            cur = g * 2 + b
            pltpu.make_async_copy(lab_hbm.at[batch, chunk_rows(0)], labv, lsem).wait()

            @pl.when(cur >= 2)
            def _drain_store():
                pltpu.make_async_copy(datv, out_hbm.at[batch, chunk_rows(0)], ssem).wait()

            def inner(j, _):
                r = j // (GPR // UNROLL)
                q = (j % (GPR // UNROLL)) * (L * UNROLL)
                labs = [labv[r, pl.ds(q + u * L, L)] for u in range(UNROLL)]
                gs = [plsc.load_gather(mean_v, [labs[u]])
                      for u in range(UNROLL)]
                for u in range(UNROLL):
                    datv[r, pl.ds(q + u * L, L)] = gs[u]
                return _
            lax.fori_loop(0, C // (L * UNROLL), inner, None)

            pltpu.async_copy(datv, out_hbm.at[batch, chunk_rows(cur)], ssem)

            nxt = cur + 2

            @pl.when(nxt < NCHUNK)
            def _start_next():
                pltpu.async_copy(lab_hbm.at[batch, chunk_rows(nxt)], labv, lsem)
        return _
    lax.fori_loop(0, NCHUNK // 2, chunk_pair2, None)

    pltpu.make_async_copy(dat0, out_hbm.at[batch, chunk_rows(0)], ssem0).wait()
    pltpu.make_async_copy(dat1, out_hbm.at[batch, chunk_rows(0)], ssem1).wait()


@jax.jit
def kernel(src, labels):
    labels = labels.astype(jnp.int32)
    mesh = plsc.VectorSubcoreMesh(
        core_axis_name="c", subcore_axis_name="s",
        num_cores=NC, num_subcores=NS)
    out = pl.kernel(
        _sc_body,
        out_type=jax.ShapeDtypeStruct((B, R, D), jnp.float32),
        mesh=mesh,
        compiler_params=pltpu.CompilerParams(needs_layout_passes=False),
        scratch_types=[
            pltpu.VMEM((NBINS,), jnp.float32),
            pltpu.VMEM((NBINS,), jnp.float32),
            pltpu.VMEM((NBINS,), jnp.float32),
            pltpu.VMEM((WPB, NBINS), jnp.float32),
            pltpu.VMEM((WPB, NBINS), jnp.float32),
            pltpu.VMEM((CR, D), jnp.int32),
            pltpu.VMEM((CR, D), jnp.int32),
            pltpu.VMEM((CR, D), jnp.float32),
            pltpu.VMEM((CR, D), jnp.float32),
            pltpu.VMEM_SHARED((NS, NBINS), jnp.float32),
            pltpu.VMEM_SHARED((NS, NBINS), jnp.float32),
            pltpu.SemaphoreType.DMA,
            pltpu.SemaphoreType.DMA,
            pltpu.SemaphoreType.DMA,
            pltpu.SemaphoreType.DMA,
        ],
    )(src, labels)
    return out


# speedup vs baseline: 1.2897x; 1.0325x over previous
import jax
import jax.numpy as jnp
from jax import lax
from jax.experimental import pallas as pl
from jax.experimental.pallas import tpu as pltpu
from jax.experimental.pallas import tpu_sc as plsc

NC = 2
NS = 16
L = 16

B = 8
R = 2048
D = 512
N = R * D
NBINS = 4096
WPB = (NC * NS) // B
ROWS_W = R // WPB
CR = 32
C = CR * D
NCHUNK = ROWS_W // CR
UNROLL = 8
GPR = D // L


def _sc_body(src_hbm, lab_hbm, out_hbm,
             sum_v, cnt_v, mean_v, red_s, red_c,
             lab0, lab1, dat0, dat1, shared_s, shared_c,
             lsem0, lsem1, ssem0, ssem1):
    c = lax.axis_index("c")
    s = lax.axis_index("s")
    batch = c * (NS // WPB) + s // WPB
    quarter = s % WPB
    row_base = quarter * ROWS_W

    bufs = ((lab0, dat0, lsem0, ssem0), (lab1, dat1, lsem1, ssem1))
    zeros = jnp.zeros((L,), jnp.float32)
    ones = jnp.ones((L,), jnp.float32)

    def chunk_rows(i):
        return pl.ds(pl.multiple_of(row_base + i * CR, CR), CR)

    def zbody(i, _):
        for u in range(UNROLL):
            ds = pl.ds(i * (L * UNROLL) + u * L, L)
            sum_v[ds] = zeros
            cnt_v[ds] = zeros
        return _
    lax.fori_loop(0, NBINS // (L * UNROLL), zbody, None)

    pltpu.async_copy(lab_hbm.at[batch, chunk_rows(0)], lab0, lsem0)
    pltpu.async_copy(src_hbm.at[batch, chunk_rows(0)], dat0, lsem0)
    pltpu.async_copy(lab_hbm.at[batch, chunk_rows(1)], lab1, lsem1)
    pltpu.async_copy(src_hbm.at[batch, chunk_rows(1)], dat1, lsem1)

    def chunk_pair1(g, _):
        for b, (labv, datv, lsem, _ssem) in enumerate(bufs):
            cur = g * 2 + b
            pltpu.make_async_copy(lab_hbm.at[batch, chunk_rows(0)], labv, lsem).wait()
            pltpu.make_async_copy(src_hbm.at[batch, chunk_rows(0)], datv, lsem).wait()

            def inner(j, _):
                r = j // (GPR // UNROLL)
                q = (j % (GPR // UNROLL)) * (L * UNROLL)
                labs = [labv[r, pl.ds(q + u * L, L)] for u in range(UNROLL)]
                xs = [datv[r, pl.ds(q + u * L, L)] for u in range(UNROLL)]
                for u in range(UNROLL):
                    plsc.addupdate_scatter(sum_v, [labs[u]], xs[u])
                    plsc.addupdate_scatter(cnt_v, [labs[u]], ones)
                return _
            lax.fori_loop(0, C // (L * UNROLL), inner, None)

            nxt = cur + 2

            @pl.when(nxt < NCHUNK)
            def _start_next():
                pltpu.async_copy(lab_hbm.at[batch, chunk_rows(nxt)], labv, lsem)
                pltpu.async_copy(src_hbm.at[batch, chunk_rows(nxt)], datv, lsem)
        return _
    lax.fori_loop(0, NCHUNK // 2, chunk_pair1, None)

    pltpu.sync_copy(sum_v, shared_s.at[s])
    pltpu.sync_copy(cnt_v, shared_c.at[s])
    plsc.subcore_barrier()
    row0 = (s // WPB) * WPB
    pltpu.sync_copy(shared_s.at[pl.ds(row0, WPB)], red_s)
    pltpu.sync_copy(shared_c.at[pl.ds(row0, WPB)], red_c)

    def mbody(i, _):
        for u in range(UNROLL):
            ds = pl.ds(i * (L * UNROLL) + u * L, L)
            ssum = red_s[0, ds] + red_s[1, ds] + red_s[2, ds] + red_s[3, ds]
            scnt = red_c[0, ds] + red_c[1, ds] + red_c[2, ds] + red_c[3, ds]
            mean_v[ds] = ssum / scnt
        return _
    lax.fori_loop(0, NBINS // (L * UNROLL), mbody, None)

    pltpu.async_copy(lab_hbm.at[batch, chunk_rows(0)], lab0, lsem0)
    pltpu.async_copy(lab_hbm.at[batch, chunk_rows(1)], lab1, lsem1)

    def chunk_pair2(g, _):
        for b, (labv, datv, lsem, ssem) in enumerate(bufs):
            cur = g * 2 + b
            pltpu.make_async_copy(lab_hbm.at[batch, chunk_rows(0)], labv, lsem).wait()

            @pl.when(cur >= 2)
            def _drain_store():
                pltpu.make_async_copy(datv, out_hbm.at[batch, chunk_rows(0)], ssem).wait()

            def inner(j, _):
                r = j // (GPR // UNROLL)
                q = (j % (GPR // UNROLL)) * (L * UNROLL)
                labs = [labv[r, pl.ds(q + u * L, L)] for u in range(UNROLL)]
                gs = [plsc.load_gather(mean_v, [labs[u]])
                      for u in range(UNROLL)]
                for u in range(UNROLL):
                    datv[r, pl.ds(q + u * L, L)] = gs[u]
                return _
            lax.fori_loop(0, C // (L * UNROLL), inner, None)

            pltpu.async_copy(datv, out_hbm.at[batch, chunk_rows(cur)], ssem)

            nxt = cur + 2

            @pl.when(nxt < NCHUNK)
            def _start_next():
                pltpu.async_copy(lab_hbm.at[batch, chunk_rows(nxt)], labv, lsem)
        return _
    lax.fori_loop(0, NCHUNK // 2, chunk_pair2, None)

    pltpu.make_async_copy(dat0, out_hbm.at[batch, chunk_rows(0)], ssem0).wait()
    pltpu.make_async_copy(dat1, out_hbm.at[batch, chunk_rows(0)], ssem1).wait()


@jax.jit
def kernel(src, labels):
    labels = labels.astype(jnp.int32)
    mesh = plsc.VectorSubcoreMesh(
        core_axis_name="c", subcore_axis_name="s",
        num_cores=NC, num_subcores=NS)
    out = pl.kernel(
        _sc_body,
        out_type=jax.ShapeDtypeStruct((B, R, D), jnp.float32),
        mesh=mesh,
        compiler_params=pltpu.CompilerParams(needs_layout_passes=False),
        scratch_types=[
            pltpu.VMEM((NBINS,), jnp.float32),
            pltpu.VMEM((NBINS,), jnp.float32),
            pltpu.VMEM((NBINS,), jnp.float32),
            pltpu.VMEM((WPB, NBINS), jnp.float32),
            pltpu.VMEM((WPB, NBINS), jnp.float32),
            pltpu.VMEM((CR, D), jnp.int32),
            pltpu.VMEM((CR, D), jnp.int32),
            pltpu.VMEM((CR, D), jnp.float32),
            pltpu.VMEM((CR, D), jnp.float32),
            pltpu.VMEM_SHARED((NS, NBINS), jnp.float32),
            pltpu.VMEM_SHARED((NS, NBINS), jnp.float32),
            pltpu.SemaphoreType.DMA,
            pltpu.SemaphoreType.DMA,
            pltpu.SemaphoreType.DMA,
            pltpu.SemaphoreType.DMA,
        ],
    )(src, labels)
    return out
